# Initial kernel scaffold; baseline (speedup 1.0000x reference)
import functools

import jax
import jax.numpy as jnp
from jax import lax
from jax.experimental import pallas as pl
from jax.experimental.pallas import tpu as pltpu
from jax.experimental.pallas import tpu_sc as plsc

N = 10000
E = 320000
D = 128
NC = 2
NS = 16
NW = NC * NS
EPW = E // NW
CH = 80
NCHUNK = EPW // CH
RPT = N // NS
ZR = 125
DW = 16

_sc_mesh = plsc.VectorSubcoreMesh(core_axis_name="c", subcore_axis_name="s")


def _zero_vmem(ref, nrows, width):
    z16 = jnp.zeros((16,), jnp.float32)

    def row(i, carry):
        for j in range(width // 16):
            ref[i, pl.ds(j * 16, 16)] = z16
        return carry

    lax.fori_loop(0, nrows, row, 0)


@functools.partial(
    pl.kernel,
    out_type=jax.ShapeDtypeStruct((NC, N, DW), jnp.float32),
    scratch_types=[
        pltpu.VMEM((CH,), jnp.int32),
        pltpu.VMEM((CH, DW), jnp.float32),
        pltpu.VMEM((ZR, DW), jnp.float32),
        pltpu.VMEM_SHARED((N, DW), jnp.float32),
    ],
    mesh=_sc_mesh,
)
def _deg_call(dst_hbm, out_hbm, dst_v, ones_v, zb_v, acc):
    c = lax.axis_index("c")
    s = lax.axis_index("s")
    w = s * NC + c
    _zero_vmem(zb_v, ZR, DW)
    one16 = jnp.ones((16,), jnp.float32)

    def fill_ones(i, carry):
        ones_v[i, pl.ds(0, 16)] = one16
        return carry

    lax.fori_loop(0, CH, fill_ones, 0)
    row0 = s * RPT
    for k in range(RPT // ZR):
        pltpu.sync_copy(zb_v, acc.at[pl.ds(row0 + k * ZR, ZR)])
    plsc.subcore_barrier()

    def chunk(k, carry):
        base = w * EPW + k * CH
        pltpu.sync_copy(dst_hbm.at[pl.ds(base, CH)], dst_v)
        pltpu.sync_copy(ones_v, acc.at[dst_v], add=True)
        return carry

    lax.fori_loop(0, NCHUNK, chunk, 0)
    plsc.subcore_barrier()
    pltpu.sync_copy(acc.at[pl.ds(row0, RPT)], out_hbm.at[c, pl.ds(row0, RPT)])


@functools.partial(
    pl.kernel,
    out_type=jax.ShapeDtypeStruct((NC, N, D), jnp.float32),
    scratch_types=[
        pltpu.VMEM((CH,), jnp.int32),
        pltpu.VMEM((CH,), jnp.int32),
        pltpu.VMEM((CH, D), jnp.float32),
        pltpu.VMEM((ZR, D), jnp.float32),
        pltpu.SemaphoreType.DMA,
        pltpu.VMEM_SHARED((N, D), jnp.float32),
    ],
    mesh=_sc_mesh,
)
def _agg_call(g_hbm, src_hbm, dst_hbm, out_hbm, src_v, dst_v, rows_v, zb_v, sem, acc):
    c = lax.axis_index("c")
    s = lax.axis_index("s")
    w = s * NC + c
    _zero_vmem(zb_v, ZR, D)
    row0 = s * RPT
    for k in range(RPT // ZR):
        pltpu.sync_copy(zb_v, acc.at[pl.ds(row0 + k * ZR, ZR)])
    plsc.subcore_barrier()

    def chunk(k, carry):
        base = w * EPW + k * CH
        pltpu.sync_copy(src_hbm.at[pl.ds(base, CH)], src_v)
        pltpu.sync_copy(dst_hbm.at[pl.ds(base, CH)], dst_v)
        pltpu.async_copy(g_hbm.at[src_v], rows_v, sem).wait()
        pltpu.sync_copy(rows_v, acc.at[dst_v], add=True)
        return carry

    lax.fori_loop(0, NCHUNK, chunk, 0)
    plsc.subcore_barrier()
    pltpu.sync_copy(acc.at[pl.ds(row0, RPT)], out_hbm.at[c, pl.ds(row0, RPT)])


_RB = 1000
_GRID = N // _RB


def _tc_first_body(x_ref, w_ref, deg_ref, g_ref, s_ref):
    sv = lax.rsqrt(deg_ref[0] + deg_ref[1] + 1.0)
    s_ref[...] = sv
    h = jnp.dot(x_ref[...], w_ref[...], preferred_element_type=jnp.float32)
    g_ref[...] = h * sv[:, 0:1]


def _tc_first(x, W1, deg2):
    return pl.pallas_call(
        _tc_first_body,
        grid=(_GRID,),
        in_specs=[
            pl.BlockSpec((_RB, D), lambda i: (i, 0)),
            pl.BlockSpec((D, D), lambda i: (0, 0)),
            pl.BlockSpec((NC, _RB, DW), lambda i: (0, i, 0)),
        ],
        out_specs=[
            pl.BlockSpec((_RB, D), lambda i: (i, 0)),
            pl.BlockSpec((_RB, DW), lambda i: (i, 0)),
        ],
        out_shape=[
            jax.ShapeDtypeStruct((N, D), jnp.float32),
            jax.ShapeDtypeStruct((N, DW), jnp.float32),
        ],
    )(x, W1, deg2)


def _tc_mid_body(agg_ref, g_ref, s_ref, b_ref, w_ref, gn_ref):
    s1 = s_ref[:, 0:1]
    z = s1 * (agg_ref[0] + agg_ref[1] + g_ref[...]) + b_ref[...]
    x2 = jnp.maximum(z, 0.0)
    gn_ref[...] = s1 * jnp.dot(x2, w_ref[...], preferred_element_type=jnp.float32)


def _tc_mid(agg, g, s16, b, Wn):
    return pl.pallas_call(
        _tc_mid_body,
        grid=(_GRID,),
        in_specs=[
            pl.BlockSpec((NC, _RB, D), lambda i: (0, i, 0)),
            pl.BlockSpec((_RB, D), lambda i: (i, 0)),
            pl.BlockSpec((_RB, DW), lambda i: (i, 0)),
            pl.BlockSpec((1, D), lambda i: (0, 0)),
            pl.BlockSpec((D, D), lambda i: (0, 0)),
        ],
        out_specs=pl.BlockSpec((_RB, D), lambda i: (i, 0)),
        out_shape=jax.ShapeDtypeStruct((N, D), jnp.float32),
    )(agg, g, s16, b, Wn)


def _tc_last_body(agg_ref, g_ref, s_ref, b_ref, out_ref):
    s1 = s_ref[:, 0:1]
    out_ref[...] = s1 * (agg_ref[0] + agg_ref[1] + g_ref[...]) + b_ref[...]


def _tc_last(agg, g, s16, b):
    return pl.pallas_call(
        _tc_last_body,
        grid=(_GRID,),
        in_specs=[
            pl.BlockSpec((NC, _RB, D), lambda i: (0, i, 0)),
            pl.BlockSpec((_RB, D), lambda i: (i, 0)),
            pl.BlockSpec((_RB, DW), lambda i: (i, 0)),
            pl.BlockSpec((1, D), lambda i: (0, 0)),
        ],
        out_specs=pl.BlockSpec((_RB, D), lambda i: (i, 0)),
        out_shape=jax.ShapeDtypeStruct((N, D), jnp.float32),
    )(agg, g, s16, b)


def kernel(x, edge_index, W1, b1, W2, b2, W3, b3):
    src = edge_index[0].astype(jnp.int32)
    dst = edge_index[1].astype(jnp.int32)
    deg2 = _deg_call(dst)
    g1, s16 = _tc_first(x, W1, deg2)
    agg1 = _agg_call(g1, src, dst)
    g2 = _tc_mid(agg1, g1, s16, b1.reshape(1, D), W2)
    agg2 = _agg_call(g2, src, dst)
    g3 = _tc_mid(agg2, g2, s16, b2.reshape(1, D), W3)
    agg3 = _agg_call(g3, src, dst)
    return _tc_last(agg3, g3, s16, b3.reshape(1, D))


# speedup vs baseline: 11.2631x; 11.2631x over previous
import functools

import jax
import jax.numpy as jnp
from jax import lax
from jax.experimental import pallas as pl
from jax.experimental.pallas import tpu as pltpu
from jax.experimental.pallas import tpu_sc as plsc

N = 10000
E = 320000
D = 128
NC = 2
NS = 16
NW = NC * NS
EPW = E // NW
CH = 80
NCHUNK = EPW // CH
NP_ = 10240
RPT = NP_ // NS
ZR = 128
DW = 16

_sc_mesh = plsc.VectorSubcoreMesh(core_axis_name="c", subcore_axis_name="s")


def _zero_vmem(ref, nrows, width):
    z16 = jnp.zeros((16,), jnp.float32)

    def row(i, carry):
        for j in range(width // 16):
            ref[i, pl.ds(j * 16, 16)] = z16
        return carry

    lax.fori_loop(0, nrows, row, 0)


@functools.partial(
    pl.kernel,
    out_type=jax.ShapeDtypeStruct((NC, NP_, D), jnp.float32),
    scratch_types=[
        pltpu.VMEM((CH,), jnp.int32),
        pltpu.VMEM((CH, D), jnp.float32),
        pltpu.VMEM((ZR, D), jnp.float32),
        pltpu.VMEM_SHARED((NP_, D), jnp.float32),
    ],
    mesh=_sc_mesh,
)
def _deg_call(dst_hbm, out_hbm, dst_v, ones_v, zb_v, acc):
    c = lax.axis_index("c")
    s = lax.axis_index("s")
    w = s * NC + c
    _zero_vmem(zb_v, ZR, D)
    one16 = jnp.ones((16,), jnp.float32)

    def fill_ones(i, carry):
        for j in range(D // 16):
            ones_v[i, pl.ds(16 * j, 16)] = one16
        return carry

    lax.fori_loop(0, CH, fill_ones, 0)
    row0 = s * RPT
    for k in range(RPT // ZR):
        pltpu.sync_copy(zb_v, acc.at[pl.ds(row0 + k * ZR, ZR)])
    plsc.subcore_barrier()

    def chunk(k, carry):
        base = w * EPW + k * CH
        pltpu.sync_copy(dst_hbm.at[pl.ds(base, CH)], dst_v)
        pltpu.sync_copy(ones_v, acc.at[dst_v], add=True)
        return carry

    lax.fori_loop(0, NCHUNK, chunk, 0)
    plsc.subcore_barrier()
    pltpu.sync_copy(acc.at[pl.ds(row0, RPT)], out_hbm.at[c, pl.ds(row0, RPT)])


@functools.partial(
    pl.kernel,
    out_type=jax.ShapeDtypeStruct((NC, NP_, D), jnp.float32),
    scratch_types=[
        pltpu.VMEM((CH,), jnp.int32),
        pltpu.VMEM((CH,), jnp.int32),
        pltpu.VMEM((CH, D), jnp.float32),
        pltpu.VMEM((ZR, D), jnp.float32),
        pltpu.SemaphoreType.DMA,
        pltpu.VMEM_SHARED((NP_, D), jnp.float32),
    ],
    mesh=_sc_mesh,
)
def _agg_call(g_hbm, src_hbm, dst_hbm, out_hbm, src_v, dst_v, rows_v, zb_v, sem, acc):
    c = lax.axis_index("c")
    s = lax.axis_index("s")
    w = s * NC + c
    _zero_vmem(zb_v, ZR, D)
    row0 = s * RPT
    for k in range(RPT // ZR):
        pltpu.sync_copy(zb_v, acc.at[pl.ds(row0 + k * ZR, ZR)])
    plsc.subcore_barrier()

    def chunk(k, carry):
        base = w * EPW + k * CH
        pltpu.sync_copy(src_hbm.at[pl.ds(base, CH)], src_v)
        pltpu.sync_copy(dst_hbm.at[pl.ds(base, CH)], dst_v)
        pltpu.async_copy(g_hbm.at[src_v], rows_v, sem).wait()
        pltpu.sync_copy(rows_v, acc.at[dst_v], add=True)
        return carry

    lax.fori_loop(0, NCHUNK, chunk, 0)
    plsc.subcore_barrier()
    pltpu.sync_copy(acc.at[pl.ds(row0, RPT)], out_hbm.at[c, pl.ds(row0, RPT)])


_RB = 1000
_GRID = N // _RB


def _tc_first_body(x_ref, w_ref, deg_ref, g_ref, s_ref):
    sv = lax.rsqrt(deg_ref[0, :, :DW] + deg_ref[1, :, :DW] + 1.0)
    s_ref[...] = sv
    h = jnp.dot(x_ref[...], w_ref[...], preferred_element_type=jnp.float32)
    g_ref[...] = h * sv[:, 0:1]


def _tc_first(x, W1, deg2):
    return pl.pallas_call(
        _tc_first_body,
        grid=(_GRID,),
        in_specs=[
            pl.BlockSpec((_RB, D), lambda i: (i, 0)),
            pl.BlockSpec((D, D), lambda i: (0, 0)),
            pl.BlockSpec((NC, _RB, D), lambda i: (0, i, 0)),
        ],
        out_specs=[
            pl.BlockSpec((_RB, D), lambda i: (i, 0)),
            pl.BlockSpec((_RB, DW), lambda i: (i, 0)),
        ],
        out_shape=[
            jax.ShapeDtypeStruct((N, D), jnp.float32),
            jax.ShapeDtypeStruct((N, DW), jnp.float32),
        ],
    )(x, W1, deg2)


def _tc_mid_body(agg_ref, g_ref, s_ref, b_ref, w_ref, gn_ref):
    s1 = s_ref[:, 0:1]
    z = s1 * (agg_ref[0] + agg_ref[1] + g_ref[...]) + b_ref[...]
    x2 = jnp.maximum(z, 0.0)
    gn_ref[...] = s1 * jnp.dot(x2, w_ref[...], preferred_element_type=jnp.float32)


def _tc_mid(agg, g, s16, b, Wn):
    return pl.pallas_call(
        _tc_mid_body,
        grid=(_GRID,),
        in_specs=[
            pl.BlockSpec((NC, _RB, D), lambda i: (0, i, 0)),
            pl.BlockSpec((_RB, D), lambda i: (i, 0)),
            pl.BlockSpec((_RB, DW), lambda i: (i, 0)),
            pl.BlockSpec((1, D), lambda i: (0, 0)),
            pl.BlockSpec((D, D), lambda i: (0, 0)),
        ],
        out_specs=pl.BlockSpec((_RB, D), lambda i: (i, 0)),
        out_shape=jax.ShapeDtypeStruct((N, D), jnp.float32),
    )(agg, g, s16, b, Wn)


def _tc_last_body(agg_ref, g_ref, s_ref, b_ref, out_ref):
    s1 = s_ref[:, 0:1]
    out_ref[...] = s1 * (agg_ref[0] + agg_ref[1] + g_ref[...]) + b_ref[...]


def _tc_last(agg, g, s16, b):
    return pl.pallas_call(
        _tc_last_body,
        grid=(_GRID,),
        in_specs=[
            pl.BlockSpec((NC, _RB, D), lambda i: (0, i, 0)),
            pl.BlockSpec((_RB, D), lambda i: (i, 0)),
            pl.BlockSpec((_RB, DW), lambda i: (i, 0)),
            pl.BlockSpec((1, D), lambda i: (0, 0)),
        ],
        out_specs=pl.BlockSpec((_RB, D), lambda i: (i, 0)),
        out_shape=jax.ShapeDtypeStruct((N, D), jnp.float32),
    )(agg, g, s16, b)


def kernel(x, edge_index, W1, b1, W2, b2, W3, b3):
    src = edge_index[0].astype(jnp.int32)
    dst = edge_index[1].astype(jnp.int32)
    deg2 = _deg_call(dst)
    g1, s16 = _tc_first(x, W1, deg2)
    agg1 = _agg_call(g1, src, dst)
    g2 = _tc_mid(agg1, g1, s16, b1.reshape(1, D), W2)
    agg2 = _agg_call(g2, src, dst)
    g3 = _tc_mid(agg2, g2, s16, b2.reshape(1, D), W3)
    agg3 = _agg_call(g3, src, dst)
    return _tc_last(agg3, g3, s16, b3.reshape(1, D))


# speedup vs baseline: 22.6363x; 2.0098x over previous
import functools

import jax
import jax.numpy as jnp
from jax import lax
from jax.experimental import pallas as pl
from jax.experimental.pallas import tpu as pltpu
from jax.experimental.pallas import tpu_sc as plsc

N = 10000
E = 320000
D = 128
NC = 2
NS = 16
NW = NC * NS
EPW = E // NW
CH = 40
NCHUNK = EPW // CH
NB = 5
G = NCHUNK // NB
CHD = 80
NCHD = EPW // CHD
GD = NCHD // NB
NP_ = 10240
RPT = NP_ // NS
DW = 16

_sc_mesh = plsc.VectorSubcoreMesh(core_axis_name="c", subcore_axis_name="s")


def _fill_vmem(ref, nrows, width, v16):
    def row(i, carry):
        for j in range(width // 16):
            ref[i, pl.ds(j * 16, 16)] = v16
        return carry

    lax.fori_loop(0, nrows, row, 0)


def _zero_acc_rows(buf, acc, row0):
    nr = buf.shape[0]
    _fill_vmem(buf, nr, D, jnp.zeros((16,), jnp.float32))
    for k in range(RPT // nr):
        pltpu.sync_copy(buf, acc.at[pl.ds(row0 + k * nr, nr)])


@functools.partial(
    pl.kernel,
    out_type=jax.ShapeDtypeStruct((NC, NP_, D), jnp.float32),
    scratch_types=(
        [pltpu.VMEM((CHD,), jnp.int32)] * NB
        + [pltpu.VMEM((CHD, D), jnp.float32)]
        + [pltpu.SemaphoreType.DMA] * (2 * NB)
        + [pltpu.VMEM_SHARED((NP_, D), jnp.float32)]
    ),
    mesh=_sc_mesh,
)
def _deg_call(dst_hbm, out_hbm, *sc):
    dsts = sc[0:NB]
    ones_v = sc[NB]
    isem = sc[NB + 1:NB + 1 + NB]
    ssem = sc[NB + 1 + NB:NB + 1 + 2 * NB]
    acc = sc[-1]
    c = lax.axis_index("c")
    s = lax.axis_index("s")
    w = s * NC + c
    row0 = s * RPT
    _zero_acc_rows(ones_v, acc, row0)
    _fill_vmem(ones_v, CHD, D, jnp.ones((16,), jnp.float32))
    plsc.subcore_barrier()

    def idx_load(k, b):
        pltpu.async_copy(dst_hbm.at[pl.ds(w * EPW + k * CHD, CHD)], dsts[b], isem[b])

    for b in range(NB):
        idx_load(b, b)

    def group(g, carry):
        sdesc = [None] * NB
        for b in range(NB):
            pltpu.make_async_copy(dst_hbm.at[pl.ds(0, CHD)], dsts[b], isem[b]).wait()
            sdesc[b] = pltpu.async_copy(ones_v, acc.at[dsts[b]], ssem[b], add=True)
        for b in range(NB):
            @pl.when(g < GD - 1)
            def _():
                sdesc[b].wait()
                idx_load((g + 1) * NB + b, b)
        return carry

    lax.fori_loop(0, GD, group, 0)
    for b in range(NB):
        pltpu.make_async_copy(out_hbm.at[c, pl.ds(0, CHD)], ones_v, ssem[b]).wait()
    plsc.subcore_barrier()
    pltpu.sync_copy(acc.at[pl.ds(row0, RPT)], out_hbm.at[c, pl.ds(row0, RPT)])


@functools.partial(
    pl.kernel,
    out_type=jax.ShapeDtypeStruct((NC, NP_, D), jnp.float32),
    scratch_types=(
        [pltpu.VMEM((CH,), jnp.int32)] * (2 * NB)
        + [pltpu.VMEM((CH, D), jnp.float32)] * NB
        + [pltpu.SemaphoreType.DMA] * (3 * NB)
        + [pltpu.VMEM_SHARED((NP_, D), jnp.float32)]
    ),
    mesh=_sc_mesh,
)
def _agg_call(g_hbm, src_hbm, dst_hbm, out_hbm, *sc):
    srcs = sc[0:NB]
    dsts = sc[NB:2 * NB]
    bufs = sc[2 * NB:3 * NB]
    isem = sc[3 * NB:4 * NB]
    gsem = sc[4 * NB:5 * NB]
    ssem = sc[5 * NB:6 * NB]
    acc = sc[-1]
    c = lax.axis_index("c")
    s = lax.axis_index("s")
    w = s * NC + c
    row0 = s * RPT
    _zero_acc_rows(bufs[0], acc, row0)
    plsc.subcore_barrier()

    def idx_load(k, b):
        base = w * EPW + k * CH
        pltpu.async_copy(src_hbm.at[pl.ds(base, CH)], srcs[b], isem[b])
        pltpu.async_copy(dst_hbm.at[pl.ds(base, CH)], dsts[b], isem[b])

    for b in range(NB):
        idx_load(b, b)

    def group(g, carry):
        gdesc, sdesc = [None] * NB, [None] * NB
        for b in range(NB):
            pltpu.make_async_copy(src_hbm.at[pl.ds(0, CH)], srcs[b], isem[b]).wait()
            pltpu.make_async_copy(src_hbm.at[pl.ds(0, CH)], dsts[b], isem[b]).wait()
            gdesc[b] = pltpu.async_copy(g_hbm.at[srcs[b]], bufs[b], gsem[b])
        for b in range(NB):
            gdesc[b].wait()
            sdesc[b] = pltpu.async_copy(bufs[b], acc.at[dsts[b]], ssem[b], add=True)
        for b in range(NB):
            @pl.when(g < G - 1)
            def _():
                sdesc[b].wait()
                idx_load((g + 1) * NB + b, b)
        return carry

    lax.fori_loop(0, G, group, 0)
    for b in range(NB):
        pltpu.make_async_copy(g_hbm.at[pl.ds(0, CH)], bufs[b], ssem[b]).wait()
    plsc.subcore_barrier()
    pltpu.sync_copy(acc.at[pl.ds(row0, RPT)], out_hbm.at[c, pl.ds(row0, RPT)])


_RB = 1000
_GRID = N // _RB


def _tc_first_body(x_ref, w_ref, deg_ref, g_ref, s_ref):
    sv = lax.rsqrt(deg_ref[0, :, :DW] + deg_ref[1, :, :DW] + 1.0)
    s_ref[...] = sv
    h = jnp.dot(x_ref[...], w_ref[...], preferred_element_type=jnp.float32)
    g_ref[...] = h * sv[:, 0:1]


def _tc_first(x, W1, deg2):
    return pl.pallas_call(
        _tc_first_body,
        grid=(_GRID,),
        in_specs=[
            pl.BlockSpec((_RB, D), lambda i: (i, 0)),
            pl.BlockSpec((D, D), lambda i: (0, 0)),
            pl.BlockSpec((NC, _RB, D), lambda i: (0, i, 0)),
        ],
        out_specs=[
            pl.BlockSpec((_RB, D), lambda i: (i, 0)),
            pl.BlockSpec((_RB, DW), lambda i: (i, 0)),
        ],
        out_shape=[
            jax.ShapeDtypeStruct((N, D), jnp.float32),
            jax.ShapeDtypeStruct((N, DW), jnp.float32),
        ],
    )(x, W1, deg2)


def _tc_mid_body(agg_ref, g_ref, s_ref, b_ref, w_ref, gn_ref):
    s1 = s_ref[:, 0:1]
    z = s1 * (agg_ref[0] + agg_ref[1] + g_ref[...]) + b_ref[...]
    x2 = jnp.maximum(z, 0.0)
    gn_ref[...] = s1 * jnp.dot(x2, w_ref[...], preferred_element_type=jnp.float32)


def _tc_mid(agg, g, s16, b, Wn):
    return pl.pallas_call(
        _tc_mid_body,
        grid=(_GRID,),
        in_specs=[
            pl.BlockSpec((NC, _RB, D), lambda i: (0, i, 0)),
            pl.BlockSpec((_RB, D), lambda i: (i, 0)),
            pl.BlockSpec((_RB, DW), lambda i: (i, 0)),
            pl.BlockSpec((1, D), lambda i: (0, 0)),
            pl.BlockSpec((D, D), lambda i: (0, 0)),
        ],
        out_specs=pl.BlockSpec((_RB, D), lambda i: (i, 0)),
        out_shape=jax.ShapeDtypeStruct((N, D), jnp.float32),
    )(agg, g, s16, b, Wn)


def _tc_last_body(agg_ref, g_ref, s_ref, b_ref, out_ref):
    s1 = s_ref[:, 0:1]
    out_ref[...] = s1 * (agg_ref[0] + agg_ref[1] + g_ref[...]) + b_ref[...]


def _tc_last(agg, g, s16, b):
    return pl.pallas_call(
        _tc_last_body,
        grid=(_GRID,),
        in_specs=[
            pl.BlockSpec((NC, _RB, D), lambda i: (0, i, 0)),
            pl.BlockSpec((_RB, D), lambda i: (i, 0)),
            pl.BlockSpec((_RB, DW), lambda i: (i, 0)),
            pl.BlockSpec((1, D), lambda i: (0, 0)),
        ],
        out_specs=pl.BlockSpec((_RB, D), lambda i: (i, 0)),
        out_shape=jax.ShapeDtypeStruct((N, D), jnp.float32),
    )(agg, g, s16, b)


def kernel(x, edge_index, W1, b1, W2, b2, W3, b3):
    src = edge_index[0].astype(jnp.int32)
    dst = edge_index[1].astype(jnp.int32)
    deg2 = _deg_call(dst)
    g1, s16 = _tc_first(x, W1, deg2)
    agg1 = _agg_call(g1, src, dst)
    g2 = _tc_mid(agg1, g1, s16, b1.reshape(1, D), W2)
    agg2 = _agg_call(g2, src, dst)
    g3 = _tc_mid(agg2, g2, s16, b2.reshape(1, D), W3)
    agg3 = _agg_call(g3, src, dst)
    return _tc_last(agg3, g3, s16, b3.reshape(1, D))


# speedup vs baseline: 22.7632x; 1.0056x over previous
import functools

import jax
import jax.numpy as jnp
from jax import lax
from jax.experimental import pallas as pl
from jax.experimental.pallas import tpu as pltpu
from jax.experimental.pallas import tpu_sc as plsc

N = 10000
E = 320000
D = 128
NC = 2
NS = 16
NW = NC * NS
EPW = E // NW
CH = 40
NCHUNK = EPW // CH
NB = 5
G = NCHUNK // NB
CHD = 80
NCHD = EPW // CHD
GD = NCHD // NB
NP_ = 10240
RPT = NP_ // NS
DW = 16

_sc_mesh = plsc.VectorSubcoreMesh(core_axis_name="c", subcore_axis_name="s")


def _fill_vmem(ref, nrows, width, v16):
    def row(i, carry):
        for j in range(width // 16):
            ref[i, pl.ds(j * 16, 16)] = v16
        return carry

    lax.fori_loop(0, nrows, row, 0)


def _zero_acc_rows(buf, acc, row0, sems):
    nr = buf.shape[0]
    ns = len(sems)
    _fill_vmem(buf, nr, D, jnp.zeros((16,), jnp.float32))
    nco = RPT // nr
    descs = [pltpu.async_copy(buf, acc.at[pl.ds(row0 + k * nr, nr)], sems[k % ns])
             for k in range(nco)]
    for d in descs:
        d.wait()


@functools.partial(
    pl.kernel,
    out_type=jax.ShapeDtypeStruct((NC, NP_, D), jnp.float32),
    scratch_types=(
        [pltpu.VMEM((CHD,), jnp.int32)] * NB
        + [pltpu.VMEM((CHD, D), jnp.float32)]
        + [pltpu.SemaphoreType.DMA] * (2 * NB)
        + [pltpu.VMEM_SHARED((NP_, D), jnp.float32)]
    ),
    mesh=_sc_mesh,
)
def _deg_call(dst_hbm, out_hbm, *sc):
    dsts = sc[0:NB]
    ones_v = sc[NB]
    isem = sc[NB + 1:NB + 1 + NB]
    ssem = sc[NB + 1 + NB:NB + 1 + 2 * NB]
    acc = sc[-1]
    c = lax.axis_index("c")
    s = lax.axis_index("s")
    w = s * NC + c
    row0 = s * RPT
    _zero_acc_rows(ones_v, acc, row0, ssem)
    _fill_vmem(ones_v, CHD, D, jnp.ones((16,), jnp.float32))
    plsc.subcore_barrier()

    def idx_load(k, b):
        pltpu.async_copy(dst_hbm.at[pl.ds(w * EPW + k * CHD, CHD)], dsts[b], isem[b])

    for b in range(NB):
        idx_load(b, b)

    def group(g, carry):
        sdesc = [None] * NB
        for b in range(NB):
            pltpu.make_async_copy(dst_hbm.at[pl.ds(0, CHD)], dsts[b], isem[b]).wait()
            sdesc[b] = pltpu.async_copy(ones_v, acc.at[dsts[b]], ssem[b], add=True)
        for b in range(NB):
            @pl.when(g < GD - 1)
            def _():
                sdesc[b].wait()
                idx_load((g + 1) * NB + b, b)
        return carry

    lax.fori_loop(0, GD, group, 0)
    for b in range(NB):
        pltpu.make_async_copy(out_hbm.at[c, pl.ds(0, CHD)], ones_v, ssem[b]).wait()
    plsc.subcore_barrier()
    pltpu.sync_copy(acc.at[pl.ds(row0, RPT)], out_hbm.at[c, pl.ds(row0, RPT)])


@functools.partial(
    pl.kernel,
    out_type=jax.ShapeDtypeStruct((NC, NP_, D), jnp.float32),
    scratch_types=(
        [pltpu.VMEM((CH,), jnp.int32)] * (2 * NB)
        + [pltpu.VMEM((CH, D), jnp.float32)] * NB
        + [pltpu.SemaphoreType.DMA] * (3 * NB)
        + [pltpu.VMEM_SHARED((NP_, D), jnp.float32)]
    ),
    mesh=_sc_mesh,
)
def _agg_call(g_hbm, src_hbm, dst_hbm, out_hbm, *sc):
    srcs = sc[0:NB]
    dsts = sc[NB:2 * NB]
    bufs = sc[2 * NB:3 * NB]
    isem = sc[3 * NB:4 * NB]
    gsem = sc[4 * NB:5 * NB]
    ssem = sc[5 * NB:6 * NB]
    acc = sc[-1]
    c = lax.axis_index("c")
    s = lax.axis_index("s")
    w = s * NC + c
    row0 = s * RPT
    _zero_acc_rows(bufs[0], acc, row0, ssem)
    plsc.subcore_barrier()

    def idx_load(k, b):
        base = w * EPW + k * CH
        pltpu.async_copy(src_hbm.at[pl.ds(base, CH)], srcs[b], isem[b])
        pltpu.async_copy(dst_hbm.at[pl.ds(base, CH)], dsts[b], isem[b])

    for b in range(NB):
        idx_load(b, b)

    def group(g, carry):
        gdesc, sdesc = [None] * NB, [None] * NB
        for b in range(NB):
            pltpu.make_async_copy(src_hbm.at[pl.ds(0, CH)], srcs[b], isem[b]).wait()
            pltpu.make_async_copy(src_hbm.at[pl.ds(0, CH)], dsts[b], isem[b]).wait()
            gdesc[b] = pltpu.async_copy(g_hbm.at[srcs[b]], bufs[b], gsem[b])
        for b in range(NB):
            gdesc[b].wait()
            sdesc[b] = pltpu.async_copy(bufs[b], acc.at[dsts[b]], ssem[b], add=True)
        for b in range(NB):
            @pl.when(g < G - 1)
            def _():
                sdesc[b].wait()
                idx_load((g + 1) * NB + b, b)
        return carry

    lax.fori_loop(0, G, group, 0)
    for b in range(NB):
        pltpu.make_async_copy(g_hbm.at[pl.ds(0, CH)], bufs[b], ssem[b]).wait()
    plsc.subcore_barrier()
    pltpu.sync_copy(acc.at[pl.ds(row0, RPT)], out_hbm.at[c, pl.ds(row0, RPT)])


_RB = 1000
_GRID = N // _RB


def _tc_mm_body(x_ref, w_ref, h_ref):
    h_ref[...] = jnp.dot(x_ref[...], w_ref[...], preferred_element_type=jnp.float32)


def _tc_mm(x, W1):
    return pl.pallas_call(
        _tc_mm_body,
        grid=(_GRID,),
        in_specs=[
            pl.BlockSpec((_RB, D), lambda i: (i, 0)),
            pl.BlockSpec((D, D), lambda i: (0, 0)),
        ],
        out_specs=pl.BlockSpec((_RB, D), lambda i: (i, 0)),
        out_shape=jax.ShapeDtypeStruct((N, D), jnp.float32),
    )(x, W1)


def _tc_scale_body(h_ref, deg_ref, g_ref, s_ref):
    sv = lax.rsqrt(deg_ref[0, :, :DW] + deg_ref[1, :, :DW] + 1.0)
    s_ref[...] = sv
    g_ref[...] = h_ref[...] * sv[:, 0:1]


def _tc_scale(h, deg2):
    return pl.pallas_call(
        _tc_scale_body,
        grid=(_GRID,),
        in_specs=[
            pl.BlockSpec((_RB, D), lambda i: (i, 0)),
            pl.BlockSpec((NC, _RB, D), lambda i: (0, i, 0)),
        ],
        out_specs=[
            pl.BlockSpec((_RB, D), lambda i: (i, 0)),
            pl.BlockSpec((_RB, DW), lambda i: (i, 0)),
        ],
        out_shape=[
            jax.ShapeDtypeStruct((N, D), jnp.float32),
            jax.ShapeDtypeStruct((N, DW), jnp.float32),
        ],
    )(h, deg2)


def _tc_mid_body(agg_ref, g_ref, s_ref, b_ref, w_ref, gn_ref):
    s1 = s_ref[:, 0:1]
    z = s1 * (agg_ref[0] + agg_ref[1] + g_ref[...]) + b_ref[...]
    x2 = jnp.maximum(z, 0.0)
    gn_ref[...] = s1 * jnp.dot(x2, w_ref[...], preferred_element_type=jnp.float32)


def _tc_mid(agg, g, s16, b, Wn):
    return pl.pallas_call(
        _tc_mid_body,
        grid=(_GRID,),
        in_specs=[
            pl.BlockSpec((NC, _RB, D), lambda i: (0, i, 0)),
            pl.BlockSpec((_RB, D), lambda i: (i, 0)),
            pl.BlockSpec((_RB, DW), lambda i: (i, 0)),
            pl.BlockSpec((1, D), lambda i: (0, 0)),
            pl.BlockSpec((D, D), lambda i: (0, 0)),
        ],
        out_specs=pl.BlockSpec((_RB, D), lambda i: (i, 0)),
        out_shape=jax.ShapeDtypeStruct((N, D), jnp.float32),
    )(agg, g, s16, b, Wn)


def _tc_last_body(agg_ref, g_ref, s_ref, b_ref, out_ref):
    s1 = s_ref[:, 0:1]
    out_ref[...] = s1 * (agg_ref[0] + agg_ref[1] + g_ref[...]) + b_ref[...]


def _tc_last(agg, g, s16, b):
    return pl.pallas_call(
        _tc_last_body,
        grid=(_GRID,),
        in_specs=[
            pl.BlockSpec((NC, _RB, D), lambda i: (0, i, 0)),
            pl.BlockSpec((_RB, D), lambda i: (i, 0)),
            pl.BlockSpec((_RB, DW), lambda i: (i, 0)),
            pl.BlockSpec((1, D), lambda i: (0, 0)),
        ],
        out_specs=pl.BlockSpec((_RB, D), lambda i: (i, 0)),
        out_shape=jax.ShapeDtypeStruct((N, D), jnp.float32),
    )(agg, g, s16, b)


def kernel(x, edge_index, W1, b1, W2, b2, W3, b3):
    src = edge_index[0].astype(jnp.int32)
    dst = edge_index[1].astype(jnp.int32)
    deg2 = _deg_call(dst)
    h1 = _tc_mm(x, W1)
    g1, s16 = _tc_scale(h1, deg2)
    agg1 = _agg_call(g1, src, dst)
    g2 = _tc_mid(agg1, g1, s16, b1.reshape(1, D), W2)
    agg2 = _agg_call(g2, src, dst)
    g3 = _tc_mid(agg2, g2, s16, b2.reshape(1, D), W3)
    agg3 = _agg_call(g3, src, dst)
    return _tc_last(agg3, g3, s16, b3.reshape(1, D))
